# baseline (speedup 1.0000x reference)
import functools

import jax
import jax.numpy as jnp
from jax import lax
from jax.experimental import pallas as pl
from jax.experimental.pallas import tpu as pltpu
from jax.experimental.pallas import tpu_sc as plsc

M_TOTAL = 2097152
B_CLONE = 262144
SIZE = M_TOTAL - B_CLONE

LANES = 128
R_TOTAL = M_TOTAL // LANES
R_HEAD = SIZE // LANES
R_TAIL = B_CLONE // LANES

RBLK = 512
GRID = R_TOTAL // RBLK
N_HEAD = R_HEAD // RBLK


def _deinterleave(x, n_comp):
    k = x.shape[1]
    row = lax.broadcasted_iota(jnp.int32, (k, LANES), 0)
    col = lax.broadcasted_iota(jnp.int32, (k, LANES), 1)
    outs = []
    for c in range(n_comp):
        p = (row == n_comp * col + c).astype(jnp.float32)
        outs.append(jnp.dot(x, p, preferred_element_type=jnp.float32,
                            precision=lax.Precision.HIGHEST))
    return outs


def _cov_body(scale_ref, gsc_ref, rot_ref, grot_ref, out_ref):
    i = pl.program_id(0)
    head = i < N_HEAD
    sc = jnp.where(head, scale_ref[...], gsc_ref[...])
    ro = jnp.where(head, rot_ref[...], grot_ref[...])

    s0, s1, s2 = _deinterleave(sc, 3)
    q0, q1, q2, q3 = _deinterleave(ro, 4)

    n2 = q0 * q0 + q1 * q1 + q2 * q2 + q3 * q3
    inv = 1.0 / jnp.maximum(jnp.sqrt(n2), 1e-12)
    w, x, y, z = q0 * inv, q1 * inv, q2 * inv, q3 * inv

    e0 = jnp.exp(s0)
    e1 = jnp.exp(s1)
    e2 = jnp.exp(s2)

    m00 = (1.0 - 2.0 * (y * y + z * z)) * e0
    m01 = (2.0 * (x * y - w * z)) * e1
    m02 = (2.0 * (x * z + w * y)) * e2
    m10 = (2.0 * (x * y + w * z)) * e0
    m11 = (1.0 - 2.0 * (x * x + z * z)) * e1
    m12 = (2.0 * (y * z - w * x)) * e2
    m20 = (2.0 * (x * z - w * y)) * e0
    m21 = (2.0 * (y * z + w * x)) * e1
    m22 = (1.0 - 2.0 * (x * x + y * y)) * e2

    c00 = m00 * m00 + m01 * m01 + m02 * m02
    c01 = m00 * m10 + m01 * m11 + m02 * m12
    c02 = m00 * m20 + m01 * m21 + m02 * m22
    c11 = m10 * m10 + m11 * m11 + m12 * m12
    c12 = m10 * m20 + m11 * m21 + m12 * m22
    c22 = m20 * m20 + m21 * m21 + m22 * m22

    rowj = lax.broadcasted_iota(jnp.int32, (LANES, 9 * LANES), 0)
    colm = lax.broadcasted_iota(jnp.int32, (LANES, 9 * LANES), 1)
    acc = None
    for a, b, comp in ((0, 0, c00), (0, 1, c01), (0, 2, c02),
                       (1, 1, c11), (1, 2, c12), (2, 2, c22)):
        sel = colm == 9 * rowj + (3 * a + b)
        if a != b:
            sel = sel | (colm == 9 * rowj + (3 * b + a))
        q = sel.astype(jnp.float32)
        term = jnp.dot(comp, q, preferred_element_type=jnp.float32,
                       precision=lax.Precision.HIGHEST)
        acc = term if acc is None else acc + term
    out_ref[...] = acc


_SC_CHUNK = 2048


def _sc_gather(scale, rotation, idx):
    info = plsc.get_sparse_core_info()
    nc, ns = info.num_cores, info.num_subcores
    nw = nc * ns
    b_per_w = B_CLONE // nw
    n_chunk = b_per_w // _SC_CHUNK
    mesh = plsc.VectorSubcoreMesh(core_axis_name="c", subcore_axis_name="s")

    scale_flat = scale.reshape(M_TOTAL * 3)
    rot_flat = rotation.reshape(M_TOTAL * 4)
    idx3 = (idx[:, None] * 3 + jnp.arange(3, dtype=idx.dtype)).reshape(-1)
    idx4 = (idx[:, None] * 4 + jnp.arange(4, dtype=idx.dtype)).reshape(-1)

    @functools.partial(
        pl.kernel,
        mesh=mesh,
        out_type=[
            jax.ShapeDtypeStruct((B_CLONE * 3,), jnp.float32),
            jax.ShapeDtypeStruct((B_CLONE * 4,), jnp.float32),
        ],
        scratch_types=[
            pltpu.VMEM((_SC_CHUNK * 3,), jnp.int32),
            pltpu.VMEM((_SC_CHUNK * 4,), jnp.int32),
            pltpu.VMEM((_SC_CHUNK * 3,), jnp.float32),
            pltpu.VMEM((_SC_CHUNK * 4,), jnp.float32),
            pltpu.SemaphoreType.DMA,
            pltpu.SemaphoreType.DMA,
        ],
    )
    def gather_k(scale_hbm, rot_hbm, idx3_hbm, idx4_hbm, gs_out, gr_out,
                 idx3_v, idx4_v, gs_v, gr_v, sem_s, sem_r):
        wid = lax.axis_index("s") * nc + lax.axis_index("c")
        base = wid * b_per_w

        def body(j, carry):
            off3 = (base + j * _SC_CHUNK) * 3
            off4 = (base + j * _SC_CHUNK) * 4
            pltpu.sync_copy(idx3_hbm.at[pl.ds(off3, _SC_CHUNK * 3)], idx3_v)
            pltpu.sync_copy(idx4_hbm.at[pl.ds(off4, _SC_CHUNK * 4)], idx4_v)
            cs = pltpu.async_copy(scale_hbm.at[idx3_v], gs_v, sem_s)
            cr = pltpu.async_copy(rot_hbm.at[idx4_v], gr_v, sem_r)
            cs.wait()
            cr.wait()
            pltpu.sync_copy(gs_v, gs_out.at[pl.ds(off3, _SC_CHUNK * 3)])
            pltpu.sync_copy(gr_v, gr_out.at[pl.ds(off4, _SC_CHUNK * 4)])
            return carry

        lax.fori_loop(0, n_chunk, body, 0)

    return gather_k(scale_flat, rot_flat, idx3, idx4)


def kernel(scale, rotation, idx):
    gs, gr = _sc_gather(scale, rotation, idx)

    scale2d = scale.reshape(R_TOTAL, 3 * LANES)
    rot2d = rotation.reshape(R_TOTAL, 4 * LANES)
    gs2d = gs.reshape(R_TAIL, 3 * LANES)
    gr2d = gr.reshape(R_TAIL, 4 * LANES)

    out2d = pl.pallas_call(
        _cov_body,
        grid=(GRID,),
        in_specs=[
            pl.BlockSpec((RBLK, 3 * LANES),
                         lambda i: (jnp.minimum(i, N_HEAD - 1), 0)),
            pl.BlockSpec((RBLK, 3 * LANES),
                         lambda i: (jnp.maximum(i - N_HEAD, 0), 0)),
            pl.BlockSpec((RBLK, 4 * LANES),
                         lambda i: (jnp.minimum(i, N_HEAD - 1), 0)),
            pl.BlockSpec((RBLK, 4 * LANES),
                         lambda i: (jnp.maximum(i - N_HEAD, 0), 0)),
        ],
        out_specs=pl.BlockSpec((RBLK, 9 * LANES), lambda i: (i, 0)),
        out_shape=jax.ShapeDtypeStruct((R_TOTAL, 9 * LANES), jnp.float32),
    )(scale2d, gs2d, rot2d, gr2d)

    return out2d.reshape(M_TOTAL, 3, 3)


# speedup vs baseline: 33.2411x; 33.2411x over previous
import functools

import jax
import jax.numpy as jnp
from jax import lax
from jax.experimental import pallas as pl
from jax.experimental.pallas import tpu as pltpu
from jax.experimental.pallas import tpu_sc as plsc

M_TOTAL = 2097152
B_CLONE = 262144
SIZE = M_TOTAL - B_CLONE

LANES = 128
R_TOTAL = M_TOTAL // LANES
R_HEAD = SIZE // LANES
R_TAIL = B_CLONE // LANES

RBLK = 512
GRID = R_TOTAL // RBLK
N_HEAD = R_HEAD // RBLK

def _cov_math(s0, s1, s2, q0, q1, q2, q3):
    n2 = q0 * q0 + q1 * q1 + q2 * q2 + q3 * q3
    inv = 1.0 / jnp.maximum(jnp.sqrt(n2), 1e-12)
    w, x, y, z = q0 * inv, q1 * inv, q2 * inv, q3 * inv

    e0 = jnp.exp(s0)
    e1 = jnp.exp(s1)
    e2 = jnp.exp(s2)

    m00 = (1.0 - 2.0 * (y * y + z * z)) * e0
    m01 = (2.0 * (x * y - w * z)) * e1
    m02 = (2.0 * (x * z + w * y)) * e2
    m10 = (2.0 * (x * y + w * z)) * e0
    m11 = (1.0 - 2.0 * (x * x + z * z)) * e1
    m12 = (2.0 * (y * z - w * x)) * e2
    m20 = (2.0 * (x * z - w * y)) * e0
    m21 = (2.0 * (y * z + w * x)) * e1
    m22 = (1.0 - 2.0 * (x * x + y * y)) * e2

    c00 = m00 * m00 + m01 * m01 + m02 * m02
    c01 = m00 * m10 + m01 * m11 + m02 * m12
    c02 = m00 * m20 + m01 * m21 + m02 * m22
    c11 = m10 * m10 + m11 * m11 + m12 * m12
    c12 = m10 * m20 + m11 * m21 + m12 * m22
    c22 = m20 * m20 + m21 * m21 + m22 * m22
    return c00, c01, c02, c11, c12, c22


def _store_planes(outr, c00, c01, c02, c11, c12, c22):
    outr[0] = c00
    outr[1] = c01
    outr[2] = c02
    outr[3] = c01
    outr[4] = c11
    outr[5] = c12
    outr[6] = c02
    outr[7] = c12
    outr[8] = c22


def _cov_body(s0r, s1r, s2r, q0r, q1r, q2r, q3r, outr):
    _store_planes(outr, *_cov_math(s0r[...], s1r[...], s2r[...],
                                   q0r[...], q1r[...], q2r[...], q3r[...]))


def _cov_tail_body(s0r, s1r, s2r, q0r, q1r, q2r, q3r, alias_r, outr):
    del alias_r
    _store_planes(outr, *_cov_math(s0r[...], s1r[...], s2r[...],
                                   q0r[...], q1r[...], q2r[...], q3r[...]))


def _sc_gather(tables, idx):
    nt = len(tables)
    info = plsc.get_sparse_core_info()
    nc, ns = info.num_cores, info.num_subcores
    nw = nc * ns
    b_per_w = B_CLONE // nw
    mesh = plsc.VectorSubcoreMesh(core_axis_name="c", subcore_axis_name="s")

    @functools.partial(
        pl.kernel,
        mesh=mesh,
        out_type=[jax.ShapeDtypeStruct((B_CLONE,), jnp.float32)
                  for _ in range(nt)],
        scratch_types=(
            [pltpu.VMEM((b_per_w,), jnp.int32)]
            + [pltpu.VMEM((b_per_w,), jnp.float32) for _ in range(nt)]
            + [pltpu.SemaphoreType.DMA for _ in range(nt)]
        ),
    )
    def gather_k(*refs):
        tbls = refs[:nt]
        idx_hbm = refs[nt]
        outs = refs[nt + 1:2 * nt + 1]
        idx_v = refs[2 * nt + 1]
        stages = refs[2 * nt + 2:3 * nt + 2]
        sems = refs[3 * nt + 2:]
        wid = lax.axis_index("s") * nc + lax.axis_index("c")
        base = wid * b_per_w

        pltpu.sync_copy(idx_hbm.at[pl.ds(base, b_per_w)], idx_v)
        copies = [
            pltpu.async_copy(tbls[t].at[idx_v], stages[t], sems[t])
            for t in range(nt)
        ]
        for t in range(nt):
            copies[t].wait()
            pltpu.sync_copy(stages[t], outs[t].at[pl.ds(base, b_per_w)])

    return gather_k(*tables, idx)


def kernel(scale, rotation, idx):
    comps = [scale[:, c] for c in range(3)] + [rotation[:, c] for c in range(4)]
    gathered = _sc_gather([c.reshape(M_TOTAL) for c in comps], idx)

    head_in = [c.reshape(R_TOTAL, LANES) for c in comps]
    tail_in = [g.reshape(R_TAIL, LANES) for g in gathered]

    in_spec = pl.BlockSpec((RBLK, LANES), lambda i: (i, 0))

    planes_head = pl.pallas_call(
        _cov_body,
        grid=(N_HEAD,),
        in_specs=[in_spec] * 7,
        out_specs=pl.BlockSpec((9, RBLK, LANES), lambda i: (0, i, 0)),
        out_shape=jax.ShapeDtypeStruct((9, R_TOTAL, LANES), jnp.float32),
    )(*head_in)

    planes = pl.pallas_call(
        _cov_tail_body,
        grid=(GRID - N_HEAD,),
        in_specs=[in_spec] * 7 + [pl.BlockSpec(memory_space=pl.ANY)],
        out_specs=pl.BlockSpec((9, RBLK, LANES), lambda i: (0, N_HEAD + i, 0)),
        out_shape=jax.ShapeDtypeStruct((9, R_TOTAL, LANES), jnp.float32),
        input_output_aliases={7: 0},
    )(*tail_in, planes_head)

    return planes.reshape(3, 3, M_TOTAL).transpose(2, 0, 1)
